# Initial kernel scaffold; baseline (speedup 1.0000x reference)
import jax
import jax.numpy as jnp
from jax.experimental import pallas as pl


def kernel(x):
    raise NotImplementedError("write your pallas kernel here")


# speedup vs baseline: 12.0167x; 12.0167x over previous
import functools

import jax
import jax.numpy as jnp
from jax.experimental import pallas as pl
from jax.experimental.pallas import tpu as pltpu

_K = 5


def _impute_body(x_blk_ref, pt_ref, sqt_ref, xzt_ref, out_ref, acc_ref, cm_ref, *, blk_r, n, d):
    i = pl.program_id(0)
    x_blk = x_blk_ref[...]
    mask_blk = jnp.isnan(x_blk)
    p_blk = (~mask_blk).astype(jnp.float32)
    xz_blk = jnp.where(mask_blk, 0.0, x_blk)
    sq_blk = xz_blk * xz_blk

    pt = pt_ref[...]
    sqt = sqt_ref[...]
    xzt = xzt_ref[...]

    s = (jnp.dot(sq_blk, pt, preferred_element_type=jnp.float32)
         + jnp.dot(p_blk, sqt, preferred_element_type=jnp.float32)
         - 2.0 * jnp.dot(xz_blk, xzt, preferred_element_type=jnp.float32))
    s = jnp.maximum(s, 0.0)
    overlap = jnp.dot(p_blk, pt, preferred_element_type=jnp.float32)
    inf = jnp.float32(jnp.inf)
    dist = jnp.where(overlap > 0,
                     (jnp.float32(d) / jnp.maximum(overlap, 1.0)) * s,
                     inf)
    row_g = jax.lax.broadcasted_iota(jnp.int32, (blk_r, n), 0) + i * blk_r
    donor = jax.lax.broadcasted_iota(jnp.int32, (blk_r, n), 1)
    dist = jnp.where(row_g == donor, inf, dist)

    col_cnt = jnp.maximum(jnp.sum(pt, axis=1, keepdims=True), 1.0)
    cm_ref[...] = jnp.sum(xzt, axis=1, keepdims=True) / col_cnt

    acc_ref[...] = jnp.zeros((blk_r, d), jnp.float32)

    def col_body(j, _):
        pj = pt_ref[pl.ds(j, 1), :]
        xzj = xzt_ref[pl.ds(j, 1), :]
        dv = jnp.where(pj > 0, dist, inf)
        ssum = jnp.zeros((blk_r, 1), jnp.float32)
        cnt = jnp.zeros((blk_r, 1), jnp.float32)
        for _r in range(_K):
            m = jnp.min(dv, axis=1, keepdims=True)
            eq = dv == m
            idxm = jnp.min(jnp.where(eq, donor, n), axis=1, keepdims=True)
            onehot = donor == idxm
            val = jnp.sum(jnp.where(onehot, xzj, 0.0), axis=1, keepdims=True)
            valid = jnp.isfinite(m).astype(jnp.float32)
            ssum = ssum + val * valid
            cnt = cnt + valid
            dv = jnp.where(onehot, inf, dv)
        cmj = cm_ref[pl.ds(j, 1), :]
        imp = jnp.where(cnt > 0, ssum / jnp.maximum(cnt, 1.0), cmj)
        lane = jax.lax.broadcasted_iota(jnp.int32, (blk_r, d), 1)
        acc_ref[...] += jnp.where(lane == j, imp, 0.0)
        return 0

    jax.lax.fori_loop(0, d, col_body, 0)
    out_ref[...] = jnp.where(mask_blk, acc_ref[...], xz_blk)


def _impute(x, blk_r=256, interpret=False):
    n, d = x.shape
    mask = jnp.isnan(x)
    xz = jnp.where(mask, 0.0, x)
    p = (~mask).astype(jnp.float32)
    pt = p.T
    xzt = xz.T
    sqt = (xz * xz).T
    grid = (n // blk_r,)
    return pl.pallas_call(
        functools.partial(_impute_body, blk_r=blk_r, n=n, d=d),
        grid=grid,
        in_specs=[
            pl.BlockSpec((blk_r, d), lambda i: (i, 0)),
            pl.BlockSpec((d, n), lambda i: (0, 0)),
            pl.BlockSpec((d, n), lambda i: (0, 0)),
            pl.BlockSpec((d, n), lambda i: (0, 0)),
        ],
        out_specs=pl.BlockSpec((blk_r, d), lambda i: (i, 0)),
        out_shape=jax.ShapeDtypeStruct((n, d), jnp.float32),
        scratch_shapes=[pltpu.VMEM((blk_r, d), jnp.float32),
                        pltpu.VMEM((d, 1), jnp.float32)],
        interpret=interpret,
    )(x, pt, sqt, xzt)


def kernel(x):
    return _impute(x)
